# Initial kernel scaffold; baseline (speedup 1.0000x reference)
import jax
import jax.numpy as jnp
from jax.experimental import pallas as pl


def kernel(word_ids, form_ids, lemma_ids, table):
    raise NotImplementedError("write your pallas kernel here")


# speedup vs baseline: 1.0037x; 1.0037x over previous
import functools

import jax
import jax.numpy as jnp
from jax import lax
from jax.experimental import pallas as pl
from jax.experimental.pallas import tpu as pltpu
from jax.experimental.pallas import tpu_sc as plsc

DIM = 64
LANES = 16
SUB = 128
NW = 32
CB = 512
NSUB = CB // SUB
NJ = 30
NROWS = NJ * NSUB


def _sc_body(table_hbm, idx_hbm, out_hbm, idx_v, gbuf, wbuf, acc, gsem, wsem):
    c = lax.axis_index("c")
    s = lax.axis_index("s")
    wid = s * 2 + c
    base = wid * CB

    pltpu.sync_copy(idx_hbm.at[wid], idx_v)

    for sb in range(NSUB):
        pltpu.async_copy(
            table_hbm.at[idx_v.at[NROWS + sb]],
            wbuf.at[pl.ds(sb * SUB, SUB), :],
            wsem,
        )

    zeros = jnp.zeros((LANES,), jnp.float32)

    def zbody(i, carry):
        for d in range(DIM // LANES):
            acc[i, pl.ds(d * LANES, LANES)] = zeros
        return carry

    lax.fori_loop(0, CB, zbody, 0)

    def rbody(r, carry):
        pltpu.async_copy(table_hbm.at[idx_v.at[r]], gbuf, gsem).wait()
        blk = lax.rem(r, NSUB) * SUB

        def ibody(i, icarry):
            row = blk + i
            for d in range(DIM // LANES):
                sl = pl.ds(d * LANES, LANES)
                plsc.addupdate(acc.at[row, sl], gbuf[i, sl])
            return icarry

        lax.fori_loop(0, SUB, ibody, 0)
        return carry

    lax.fori_loop(0, NROWS, rbody, 0)

    for sb in range(NSUB):
        pltpu.make_async_copy(
            table_hbm.at[idx_v.at[NROWS + sb]],
            wbuf.at[pl.ds(sb * SUB, SUB), :],
            wsem,
        ).wait()

    def fbody(i, carry):
        for d in range(DIM // LANES):
            sl = pl.ds(d * LANES, LANES)
            acc[i, sl] = (acc[i, sl] + wbuf[i, sl] * 15.0) * (1.0 / 45.0)
        return carry

    lax.fori_loop(0, CB, fbody, 0)

    pltpu.sync_copy(acc, out_hbm.at[pl.ds(base, CB), :])


def kernel(word_ids, form_ids, lemma_ids, table):
    B = word_ids.shape[0]
    fl = jnp.concatenate(
        [form_ids.reshape(B, 15), lemma_ids.reshape(B, 15)], axis=1
    ).astype(jnp.int32)
    flt = (
        fl.T.reshape(NJ, NW, NSUB, SUB)
        .transpose(1, 0, 2, 3)
        .reshape(NW, NROWS, SUB)
    )
    wv = word_ids.astype(jnp.int32).reshape(NW, NSUB, SUB)
    idx_all = jnp.concatenate([flt, wv], axis=1)

    mesh = plsc.VectorSubcoreMesh(core_axis_name="c", subcore_axis_name="s")
    run = functools.partial(
        pl.kernel,
        mesh=mesh,
        out_type=jax.ShapeDtypeStruct((B, DIM), jnp.float32),
        scratch_types=[
            pltpu.VMEM((NROWS + NSUB, SUB), jnp.int32),
            pltpu.VMEM((SUB, DIM), jnp.float32),
            pltpu.VMEM((CB, DIM), jnp.float32),
            pltpu.VMEM((CB, DIM), jnp.float32),
            pltpu.SemaphoreType.DMA,
            pltpu.SemaphoreType.DMA,
        ],
        compiler_params=pltpu.CompilerParams(use_tc_tiling_on_sc=False),
    )(_sc_body)
    return run(table, idx_all)


# speedup vs baseline: 1.3446x; 1.3397x over previous
import functools

import jax
import jax.numpy as jnp
from jax import lax
from jax.experimental import pallas as pl
from jax.experimental.pallas import tpu as pltpu
from jax.experimental.pallas import tpu_sc as plsc

DIM = 64
LANES = 16
SUB = 128
NW = 32
CB = 512
NSUB = CB // SUB
NJ = 30
NROWS = NJ * NSUB


def _sc_body(table_hbm, idx_hbm, out_hbm, idx_v, gbufs, wbuf, acc, sems, wsem):
    c = lax.axis_index("c")
    s = lax.axis_index("s")
    wid = s * 2 + c
    base = wid * CB

    pltpu.sync_copy(idx_hbm.at[wid], idx_v)

    for sb in range(NSUB):
        pltpu.async_copy(
            table_hbm.at[idx_v.at[NROWS + sb]],
            wbuf.at[pl.ds(sb * SUB, SUB), :],
            wsem,
        )

    zeros = jnp.zeros((LANES,), jnp.float32)

    @plsc.parallel_loop(0, CB, 1, unroll=8)
    def _zero(i):
        for d in range(DIM // LANES):
            acc[i, pl.ds(d * LANES, LANES)] = zeros

    def _start(r, b):
        pltpu.async_copy(table_hbm.at[idx_v.at[r]], gbufs.at[b], sems.at[b])

    def _drain(b):
        pltpu.make_async_copy(
            table_hbm.at[idx_v.at[0]], gbufs.at[b], sems.at[b]
        ).wait()

    def _accum(r, b):
        blk = lax.rem(r, NSUB) * SUB

        @plsc.parallel_loop(0, SUB, 1, unroll=8)
        def _(i):
            row = blk + i
            for d in range(DIM // LANES):
                sl = pl.ds(d * LANES, LANES)
                plsc.addupdate(acc.at[row, sl], gbufs[b, i, sl])

    _start(0, 0)

    def tbody(t, carry):
        r0 = 2 * t
        _start(r0 + 1, 1)
        _drain(0)
        _accum(r0, 0)

        @pl.when(r0 + 2 < NROWS)
        def _():
            _start(r0 + 2, 0)

        _drain(1)
        _accum(r0 + 1, 1)
        return carry

    lax.fori_loop(0, NROWS // 2, tbody, 0)

    for sb in range(NSUB):
        pltpu.make_async_copy(
            table_hbm.at[idx_v.at[NROWS + sb]],
            wbuf.at[pl.ds(sb * SUB, SUB), :],
            wsem,
        ).wait()

    @plsc.parallel_loop(0, CB, 1, unroll=8)
    def _final(i):
        for d in range(DIM // LANES):
            sl = pl.ds(d * LANES, LANES)
            acc[i, sl] = (acc[i, sl] + wbuf[i, sl] * 15.0) * (1.0 / 45.0)

    pltpu.sync_copy(acc, out_hbm.at[pl.ds(base, CB), :])


def kernel(word_ids, form_ids, lemma_ids, table):
    B = word_ids.shape[0]
    fl = jnp.concatenate(
        [form_ids.reshape(B, 15), lemma_ids.reshape(B, 15)], axis=1
    ).astype(jnp.int32)
    flt = (
        fl.T.reshape(NJ, NW, NSUB, SUB)
        .transpose(1, 0, 2, 3)
        .reshape(NW, NROWS, SUB)
    )
    wv = word_ids.astype(jnp.int32).reshape(NW, NSUB, SUB)
    idx_all = jnp.concatenate([flt, wv], axis=1)

    mesh = plsc.VectorSubcoreMesh(core_axis_name="c", subcore_axis_name="s")
    run = functools.partial(
        pl.kernel,
        mesh=mesh,
        out_type=jax.ShapeDtypeStruct((B, DIM), jnp.float32),
        scratch_types=[
            pltpu.VMEM((NROWS + NSUB, SUB), jnp.int32),
            pltpu.VMEM((2, SUB, DIM), jnp.float32),
            pltpu.VMEM((CB, DIM), jnp.float32),
            pltpu.VMEM((CB, DIM), jnp.float32),
            pltpu.SemaphoreType.DMA((2,)),
            pltpu.SemaphoreType.DMA,
        ],
        compiler_params=pltpu.CompilerParams(use_tc_tiling_on_sc=False),
    )(_sc_body)
    return run(table, idx_all)
